# baseline (speedup 1.0000x reference)
import functools

import jax
import jax.numpy as jnp
from jax import lax
from jax.experimental import pallas as pl
from jax.experimental.pallas import tpu as pltpu
from jax.experimental.pallas import tpu_sc as plsc

N = 10000
E = 160000
IN = 128
H = 128
ED = 16
NL = 3
G = 512
LCAP = 128
NC, NS = 2, 16
NW = NC * NS
NEG = -3.4e38

_mesh = plsc.VectorSubcoreMesh(core_axis_name="c", subcore_axis_name="s",
                               num_cores=NC, num_subcores=NS)


def _wid():
    return lax.axis_index("s") * NC + lax.axis_index("c")


def _ext(vec16, j):
    return vec16[j]


NT6 = 320
ACC6 = (NT6 + 1) * H


def _seg_kernel(hb, perm_pad, srcs_pad, segst_pad, hatom, xnew,
                accs, accm, startsw, idxb, segb, rowb, hab, sem):
    w = _wid()
    n0 = w * NT6

    def zero(i, _):
        accs[pl.ds(i * 16, 16)] = jnp.zeros((16,), jnp.float32)
        accm[pl.ds(i * 16, 16)] = jnp.zeros((16,), jnp.float32)
        return 0
    lax.fori_loop(0, ACC6 // 16, zero, 0)

    pltpu.sync_copy(segst_pad.at[pl.ds(n0, 336)], startsw)
    e0 = _ext(startsw[pl.ds(0, 16)], 0)
    e1 = _ext(startsw[pl.ds(320, 16)], 0)
    e0a = (e0 // 16) * 16
    nch = (e1 - e0a + 15) // 16

    def chunk(c, _):
        base = e0a + c * 16
        pltpu.sync_copy(perm_pad.at[pl.ds(base, 16)], idxb)
        pltpu.sync_copy(srcs_pad.at[pl.ds(base, 16)], segb)
        pltpu.async_copy(hb.at[idxb], rowb, sem).wait()
        sv = segb[...]
        for j in range(16):
            s = _ext(sv, j) - n0
            inr = jnp.logical_and(s >= 0, s < NT6)
            off = jnp.where(inr, s, NT6) * H
            for f in range(8):
                v = rowb[j, pl.ds(f * 16, 16)]
                sl = pl.ds(off + f * 16, 16)
                accs[sl] = accs[sl] + v
                accm[sl] = jnp.maximum(accm[sl], v)
        return 0
    lax.fori_loop(0, nch, chunk, 0)

    def group(g, _):
        node0 = n0 + g * 16

        @pl.when(node0 < N)
        def _():
            pltpu.sync_copy(hatom.at[pl.ds(node0, 16)], hab)
            for j in range(16):
                off = g * (16 * H) + j * H
                for f in range(8):
                    sl = pl.ds(off + f * 16, 16)
                    hab[j, pl.ds(f * 16, 16)] = (
                        hab[j, pl.ds(f * 16, 16)] + accs[sl] * accm[sl])
            pltpu.sync_copy(hab, xnew.at[pl.ds(node0, 16)])
        return 0
    lax.fori_loop(0, NT6 // 16, group, 0)


@jax.jit
def _seg_call(hb, perm_pad, srcs_pad, segst_pad, hatom):
    f = pl.kernel(
        _seg_kernel,
        out_type=jax.ShapeDtypeStruct((N, H), jnp.float32),
        mesh=_mesh,
        scratch_types=[
            pltpu.VMEM((ACC6,), jnp.float32),
            pltpu.VMEM((ACC6,), jnp.float32),
            pltpu.VMEM((336,), jnp.int32),
            pltpu.VMEM((16,), jnp.int32),
            pltpu.VMEM((16,), jnp.int32),
            pltpu.VMEM((16, H), jnp.float32),
            pltpu.VMEM((16, H), jnp.float32),
            pltpu.SemaphoreType.DMA,
        ],
    )
    return f(hb, perm_pad, srcs_pad, segst_pad, hatom)


EC5 = 40
EPT5 = E // NW


def _edge_kernel(xnew, hb, src, dst, out, ia, ib, xa, xb, sem):
    w = _wid()
    base_t = w * EPT5

    def chunk(c, _):
        base = base_t + c * EC5
        pltpu.sync_copy(src.at[pl.ds(base, EC5)], ia)
        pltpu.sync_copy(dst.at[pl.ds(base, EC5)], ib)
        cp1 = pltpu.async_copy(xnew.at[ia], xa, sem)
        cp1.wait()
        cp2 = pltpu.async_copy(hb.at[ib], xb, sem)
        cp2.wait()
        for r in range(EC5):
            for f in range(8):
                sl = pl.ds(f * 16, 16)
                xa[r, sl] = xa[r, sl] - xb[r, sl]
        pltpu.sync_copy(xa, out.at[pl.ds(base, EC5)])
        return 0
    lax.fori_loop(0, EPT5 // EC5, chunk, 0)


@jax.jit
def _edge_call(xnew, hb, src, dst):
    f = pl.kernel(
        _edge_kernel,
        out_type=jax.ShapeDtypeStruct((E, H), jnp.float32),
        mesh=_mesh,
        scratch_types=[
            pltpu.VMEM((EC5,), jnp.int32),
            pltpu.VMEM((EC5,), jnp.int32),
            pltpu.VMEM((EC5, H), jnp.float32),
            pltpu.VMEM((EC5, H), jnp.float32),
            pltpu.SemaphoreType.DMA,
        ],
    )
    return f(xnew, hb, src, dst)


RC1 = 64
NCH1 = LCAP * G // RC1 // NW


def _pad_kernel(msg, pad_idx, lmax_v, mpad, idxb, rowb, lmb, sem):
    w = _wid()
    pltpu.sync_copy(lmax_v.at[pl.ds(0, 16)], lmb)
    lmax = lmb[...][0]

    def chunk(c, _):
        row0 = (c * NW + w) * RC1
        t0 = row0 // G

        @pl.when(t0 < lmax)
        def _():
            pltpu.sync_copy(pad_idx.at[pl.ds(row0, RC1)], idxb)
            pltpu.async_copy(msg.at[idxb], rowb, sem).wait()
            pltpu.sync_copy(rowb, mpad.at[pl.ds(row0, RC1)])
        return 0
    lax.fori_loop(0, NCH1, chunk, 0)


@jax.jit
def _pad_call(msg_ext, pad_idx, lmax_v):
    f = pl.kernel(
        _pad_kernel,
        out_type=jax.ShapeDtypeStruct((LCAP * G, H), jnp.float32),
        mesh=_mesh,
        scratch_types=[
            pltpu.VMEM((RC1,), jnp.int32),
            pltpu.VMEM((RC1, H), jnp.float32),
            pltpu.VMEM((16,), jnp.int32),
            pltpu.SemaphoreType.DMA,
        ],
    )
    return f(msg_ext, pad_idx, lmax_v)


def _h0_kernel(hmat, gst_pad, h0, win, rowb, h0b, sem):
    w = _wid()
    pltpu.sync_copy(gst_pad.at[pl.ds(w * 16, 24)], win)
    lo = win[pl.ds(0, 16)]
    hi = win[pl.ds(8, 16)]
    for j in range(16):
        g0 = _ext(lo, j)
        g1 = _ext(lo, j + 1) if j < 15 else _ext(hi, 8)
        g0a = (g0 // 8) * 8
        nch = (g1 - g0a + 7) // 8

        def chunk(c, acc):
            p0 = g0a + c * 8
            pltpu.sync_copy(hmat.at[pl.ds(p0, 8)], rowb)
            for i in range(8):
                p = p0 + i
                valid = jnp.logical_and(p >= g0, p < g1)
                acc = tuple(
                    jnp.maximum(acc[f],
                                jnp.where(valid, rowb[i, pl.ds(f * 16, 16)],
                                          jnp.float32(NEG)))
                    for f in range(8))
            return acc
        acc0 = tuple(jnp.full((16,), NEG, jnp.float32) for _ in range(8))
        acc = lax.fori_loop(0, nch, chunk, acc0)
        for f in range(8):
            h0b[j, pl.ds(f * 16, 16)] = jnp.where(acc[f] < -1e37,
                                                  jnp.float32(0.0), acc[f])
    pltpu.sync_copy(h0b, h0.at[pl.ds(w * 16, 16)])


@jax.jit
def _h0_call(hmat, gst_pad):
    f = pl.kernel(
        _h0_kernel,
        out_type=jax.ShapeDtypeStruct((G, H), jnp.float32),
        mesh=_mesh,
        scratch_types=[
            pltpu.VMEM((24,), jnp.int32),
            pltpu.VMEM((8, H), jnp.float32),
            pltpu.VMEM((16, H), jnp.float32),
            pltpu.SemaphoreType.DMA,
        ],
    )
    return f(hmat, gst_pad)


OROWS = LCAP * G + N + 16
RPT2 = 320


def _unpad_kernel(of, ob, oidx, outf, outb, idxb, ra, rb, sem):
    w = _wid()

    def chunk(c, _):
        row0 = w * RPT2 + c * 16

        @pl.when(row0 < N)
        def _():
            pltpu.sync_copy(oidx.at[pl.ds(row0, 16)], idxb)
            pltpu.async_copy(of.at[idxb], ra, sem).wait()
            pltpu.async_copy(ob.at[idxb], rb, sem).wait()
            pltpu.sync_copy(ra, outf.at[pl.ds(row0, 16)])
            pltpu.sync_copy(rb, outb.at[pl.ds(row0, 16)])
        return 0
    lax.fori_loop(0, RPT2 // 16, chunk, 0)


@jax.jit
def _unpad_call(of, ob, oidx):
    f = pl.kernel(
        _unpad_kernel,
        out_type=(jax.ShapeDtypeStruct((N, H), jnp.float32),
                  jax.ShapeDtypeStruct((N, H), jnp.float32)),
        mesh=_mesh,
        scratch_types=[
            pltpu.VMEM((16,), jnp.int32),
            pltpu.VMEM((16, H), jnp.float32),
            pltpu.VMEM((16, H), jnp.float32),
            pltpu.SemaphoreType.DMA,
        ],
    )
    return f(of, ob, oidx)


def _mm_relu_body(a_ref, w_ref, b_ref, o_ref):
    o_ref[...] = jax.nn.relu(
        jnp.dot(a_ref[...], w_ref[...], preferred_element_type=jnp.float32)
        + b_ref[...])


def _mm_relu(a, w, b, blk):
    m, k = a.shape
    h = w.shape[1]
    return pl.pallas_call(
        _mm_relu_body,
        grid=(m // blk,),
        in_specs=[
            pl.BlockSpec((blk, k), lambda i: (i, 0)),
            pl.BlockSpec((k, h), lambda i: (0, 0)),
            pl.BlockSpec((1, h), lambda i: (0, 0)),
        ],
        out_specs=pl.BlockSpec((blk, h), lambda i: (i, 0)),
        out_shape=jax.ShapeDtypeStruct((m, h), jnp.float32),
    )(a, w, b.reshape(1, h))


def _readout_body(am_ref, ha_ref, xp_ref, w1, w2, w3, bl, gb, h_ref, msg_ref):
    hv = (jnp.dot(am_ref[...], w1[...], preferred_element_type=jnp.float32)
          + jnp.dot(ha_ref[...], w2[...], preferred_element_type=jnp.float32)
          + jnp.dot(xp_ref[...], w3[...], preferred_element_type=jnp.float32)
          + bl[...])
    h_ref[...] = hv
    msg_ref[...] = jax.nn.relu(hv + gb[...])


def _readout(am, ha, xp, W_l, b_l, gru_bias):
    blk = 1000
    w1, w2, w3 = W_l[:H], W_l[H:2 * H], W_l[2 * H:]
    return pl.pallas_call(
        _readout_body,
        grid=(N // blk,),
        in_specs=[
            pl.BlockSpec((blk, H), lambda i: (i, 0)),
            pl.BlockSpec((blk, H), lambda i: (i, 0)),
            pl.BlockSpec((blk, H), lambda i: (i, 0)),
            pl.BlockSpec((H, H), lambda i: (0, 0)),
            pl.BlockSpec((H, H), lambda i: (0, 0)),
            pl.BlockSpec((H, H), lambda i: (0, 0)),
            pl.BlockSpec((1, H), lambda i: (0, 0)),
            pl.BlockSpec((1, H), lambda i: (0, 0)),
        ],
        out_specs=[pl.BlockSpec((blk, H), lambda i: (i, 0)),
                   pl.BlockSpec((blk, H), lambda i: (i, 0))],
        out_shape=[jax.ShapeDtypeStruct((N, H), jnp.float32),
                   jax.ShapeDtypeStruct((N, H), jnp.float32)],
    )(am, ha, xp, w1, w2, w3, b_l.reshape(1, H), gru_bias.reshape(1, H))


def _final_body(a_ref, b_ref, w1, w2, bo, o_ref):
    o_ref[...] = jax.nn.relu(
        jnp.dot(a_ref[...], w1[...], preferred_element_type=jnp.float32)
        + jnp.dot(b_ref[...], w2[...], preferred_element_type=jnp.float32)
        + bo[...])


def _final(outf, outb, W_o, b_o):
    blk = 1000
    return pl.pallas_call(
        _final_body,
        grid=(N // blk,),
        in_specs=[
            pl.BlockSpec((blk, H), lambda i: (i, 0)),
            pl.BlockSpec((blk, H), lambda i: (i, 0)),
            pl.BlockSpec((H, H), lambda i: (0, 0)),
            pl.BlockSpec((H, H), lambda i: (0, 0)),
            pl.BlockSpec((1, H), lambda i: (0, 0)),
        ],
        out_specs=pl.BlockSpec((blk, H), lambda i: (i, 0)),
        out_shape=jax.ShapeDtypeStruct((N, H), jnp.float32),
    )(outf, outb, W_o[:H], W_o[H:], b_o.reshape(1, H))


def _gru_body(lmax_ref, starts_ref, counts_ref, mpad, msg_ext, wih, whh,
              bih, bhh, h0_ref, outf, outb, h_ref, xt_ref, sem0, sem1):
    lmax = lmax_ref[0]

    for d in (0, 1):
        out_ref = outf if d == 0 else outb
        h_ref[...] = h0_ref[...]

        def step(i, _):
            t = i if d == 0 else lmax - 1 - i

            @pl.when(t < LCAP)
            def _():
                cp = pltpu.make_async_copy(
                    mpad.at[pl.ds(t * G, G)], xt_ref, sem0)
                cp.start()
                cp.wait()

            @pl.when(t >= LCAP)
            def _():
                def gb(g, _):
                    idx = jnp.where(t < counts_ref[g],
                                    starts_ref[g] + t, N)
                    cp = pltpu.make_async_copy(
                        msg_ext.at[pl.ds(idx, 1)], xt_ref.at[pl.ds(g, 1)],
                        sem0)
                    cp.start()
                    cp.wait()
                    return 0
                lax.fori_loop(0, G, gb, 0)

            xt = xt_ref[...]
            hprev = h_ref[...]
            gi = jnp.dot(xt, wih[d], preferred_element_type=jnp.float32) \
                + bih[d]
            gh = jnp.dot(hprev, whh[d], preferred_element_type=jnp.float32) \
                + bhh[d]
            r = jax.nn.sigmoid(gi[:, :H] + gh[:, :H])
            z = jax.nn.sigmoid(gi[:, H:2 * H] + gh[:, H:2 * H])
            n = jnp.tanh(gi[:, 2 * H:] + r * gh[:, 2 * H:])
            hn = (1.0 - z) * n + z * hprev
            h_ref[...] = hn

            @pl.when(t < LCAP)
            def _():
                cp = pltpu.make_async_copy(
                    h_ref, out_ref.at[pl.ds(t * G, G)], sem1)
                cp.start()
                cp.wait()

            @pl.when(t >= LCAP)
            def _():
                def sb(g, _):
                    node = jnp.where(t < counts_ref[g],
                                     starts_ref[g] + t, N)
                    cp = pltpu.make_async_copy(
                        h_ref.at[pl.ds(g, 1)],
                        out_ref.at[pl.ds(LCAP * G + node, 1)], sem1)
                    cp.start()
                    cp.wait()
                    return 0
                lax.fori_loop(0, G, sb, 0)
            return 0
        lax.fori_loop(0, lmax, step, 0)


def _gru(lmax_v, gstarts, counts, mpad, msg_ext, wihT, whhT, bih, bhh, h0):
    return pl.pallas_call(
        _gru_body,
        in_specs=[
            pl.BlockSpec(memory_space=pltpu.SMEM),
            pl.BlockSpec(memory_space=pltpu.SMEM),
            pl.BlockSpec(memory_space=pltpu.SMEM),
            pl.BlockSpec(memory_space=pl.ANY),
            pl.BlockSpec(memory_space=pl.ANY),
            pl.BlockSpec(memory_space=pltpu.VMEM),
            pl.BlockSpec(memory_space=pltpu.VMEM),
            pl.BlockSpec(memory_space=pltpu.VMEM),
            pl.BlockSpec(memory_space=pltpu.VMEM),
            pl.BlockSpec(memory_space=pltpu.VMEM),
        ],
        out_specs=[pl.BlockSpec(memory_space=pl.ANY),
                   pl.BlockSpec(memory_space=pl.ANY)],
        out_shape=[jax.ShapeDtypeStruct((OROWS, H), jnp.float32),
                   jax.ShapeDtypeStruct((OROWS, H), jnp.float32)],
        scratch_shapes=[
            pltpu.VMEM((G, H), jnp.float32),
            pltpu.VMEM((G, H), jnp.float32),
            pltpu.SemaphoreType.DMA,
            pltpu.SemaphoreType.DMA,
        ],
    )(lmax_v, gstarts, counts, mpad, msg_ext, wihT, whhT, bih, bhh, h0)


def kernel(x, edge_index, edge_attr, batch, W_a, b_a, W_b, b_b, Ws, bs, W_l,
           b_l, gru_bias, Wih, Whh, bih, bhh, W_o, b_o):
    src = edge_index[0]
    dst = edge_index[1]

    perm = jnp.argsort(src).astype(jnp.int32)
    src_sorted = src[perm]
    seg_starts = jnp.searchsorted(src_sorted,
                                  jnp.arange(N + 1, dtype=jnp.int32)
                                  ).astype(jnp.int32)
    segst_pad = jnp.concatenate(
        [seg_starts, jnp.full((10336 - (N + 1),), E, jnp.int32)])
    perm_pad = jnp.concatenate([perm, jnp.zeros((16,), jnp.int32)])
    srcs_pad = jnp.concatenate(
        [src_sorted, jnp.full((16,), 1 << 30, jnp.int32)])

    gstarts = jnp.searchsorted(batch, jnp.arange(G + 1, dtype=jnp.int32)
                               ).astype(jnp.int32)
    counts = gstarts[1:] - gstarts[:-1]
    lmax = jnp.max(counts)
    lmax_v = jnp.full((16,), lmax, jnp.int32)
    gst_pad = jnp.concatenate([gstarts, jnp.full((15,), N, jnp.int32)])

    tt = jnp.arange(LCAP, dtype=jnp.int32)[:, None]
    gg = jnp.arange(G, dtype=jnp.int32)[None, :]
    pad_idx = jnp.where(tt < counts[None, :], gstarts[None, :-1] + tt,
                        N).reshape(-1).astype(jnp.int32)

    node = jnp.arange(N, dtype=jnp.int32)
    tnode = node - gstarts[batch]
    oidx = jnp.where(tnode < LCAP, tnode * G + batch,
                     LCAP * G + node).astype(jnp.int32)

    x_proj = _mm_relu(x, W_a, b_a, 1000)
    h_bond = _mm_relu(edge_attr, W_b, b_b, 2000)

    h_atom = x_proj
    for l in range(NL - 1):
        x_new = _seg_call(h_bond, perm_pad, srcs_pad, segst_pad, h_atom)
        bond_embed = _edge_call(x_new, h_bond, src, dst)
        h_bond = _mm_relu(bond_embed, Ws[l], bs[l], 2000)
        h_atom = x_new
    aggr = _seg_call(h_bond, perm_pad, srcs_pad, segst_pad, h_atom)

    h, message = _readout(aggr, h_atom, x_proj, W_l, b_l, gru_bias)
    msg_ext = jnp.concatenate([message, jnp.zeros((16, H), jnp.float32)])

    h0 = _h0_call(h, gst_pad)
    mpad = _pad_call(msg_ext, pad_idx, lmax_v)
    of, ob = _gru(lmax.reshape(1), gstarts[:-1], counts, mpad, msg_ext,
                  jnp.transpose(Wih, (0, 2, 1)), jnp.transpose(Whh, (0, 2, 1)),
                  bih[:, None, :], bhh[:, None, :], h0)
    out_f, out_b = _unpad_call(of, ob, oidx)
    return _final(out_f, out_b, W_o, b_o)


# speedup vs baseline: 3.4643x; 3.4643x over previous
import functools

import jax
import jax.numpy as jnp
from jax import lax
from jax.experimental import pallas as pl
from jax.experimental.pallas import tpu as pltpu
from jax.experimental.pallas import tpu_sc as plsc

N = 10000
E = 160000
IN = 128
H = 128
ED = 16
NL = 3
G = 512
LCAP = 128
NC, NS = 2, 16
NW = NC * NS
NEG = -3.4e38

_mesh = plsc.VectorSubcoreMesh(core_axis_name="c", subcore_axis_name="s",
                               num_cores=NC, num_subcores=NS)


def _wid():
    return lax.axis_index("s") * NC + lax.axis_index("c")


def _ext(vec16, j):
    return vec16[j]


NT6 = 320
ACC6 = (NT6 + 1) * H


BLK6 = 512
SUB6 = 32


def _seg_kernel(hb, perm_pad, srcs_pad, tb, hatom, xnew,
                accs, accm, startsw, idxblk, segblk, idxsub, rowb, hab, sem):
    w = _wid()
    n0 = w * NT6

    def zero(i, _):
        accs[pl.ds(i * 16, 16)] = jnp.zeros((16,), jnp.float32)
        accm[pl.ds(i * 16, 16)] = jnp.zeros((16,), jnp.float32)
        return 0
    lax.fori_loop(0, ACC6 // 16, zero, 0)

    pltpu.sync_copy(tb.at[pl.ds(w * 16, 16)], startsw)
    bv = startsw[...]
    e0 = bv[0]
    e1 = bv[1]
    e0a = (e0 // 16) * 16
    nblk = (e1 - e0a + BLK6 - 1) // BLK6

    def block(b, _):
        base = e0a + b * BLK6
        pltpu.sync_copy(perm_pad.at[pl.ds(base, BLK6)], idxblk)
        pltpu.sync_copy(srcs_pad.at[pl.ds(base, BLK6)], segblk)
        nsub = jnp.minimum((e1 - base + SUB6 - 1) // SUB6, BLK6 // SUB6)

        def sub(c, _):
            for q in range(SUB6 // 16):
                idxsub[pl.ds(q * 16, 16)] = idxblk[pl.ds(c * SUB6 + q * 16,
                                                         16)]
            pltpu.async_copy(hb.at[idxsub], rowb, sem).wait()
            for jq in range(SUB6 // 16):
                sv = segblk[pl.ds(c * SUB6 + jq * 16, 16)]
                for j in range(16):
                    s = _ext(sv, j) - n0
                    inr = jnp.logical_and(s >= 0, s < NT6)
                    off = jnp.where(inr, s, NT6) * H
                    for f in range(8):
                        v = rowb[jq * 16 + j, pl.ds(f * 16, 16)]
                        sl = pl.ds(off + f * 16, 16)
                        accs[sl] = accs[sl] + v
                        accm[sl] = jnp.maximum(accm[sl], v)
            return 0
        lax.fori_loop(0, nsub, sub, 0)
        return 0
    lax.fori_loop(0, nblk, block, 0)

    def group(g, _):
        node0 = n0 + g * 16

        @pl.when(node0 < N)
        def _():
            pltpu.sync_copy(hatom.at[pl.ds(node0, 16)], hab)
            for j in range(16):
                off = g * (16 * H) + j * H
                for f in range(8):
                    sl = pl.ds(off + f * 16, 16)
                    hab[j, pl.ds(f * 16, 16)] = (
                        hab[j, pl.ds(f * 16, 16)] + accs[sl] * accm[sl])
            pltpu.sync_copy(hab, xnew.at[pl.ds(node0, 16)])
        return 0
    lax.fori_loop(0, NT6 // 16, group, 0)


@jax.jit
def _seg_call(hb, perm_pad, srcs_pad, tb, hatom):
    f = pl.kernel(
        _seg_kernel,
        out_type=jax.ShapeDtypeStruct((N, H), jnp.float32),
        mesh=_mesh,
        scratch_types=[
            pltpu.VMEM((ACC6,), jnp.float32),
            pltpu.VMEM((ACC6,), jnp.float32),
            pltpu.VMEM((16,), jnp.int32),
            pltpu.VMEM((BLK6,), jnp.int32),
            pltpu.VMEM((BLK6,), jnp.int32),
            pltpu.VMEM((SUB6,), jnp.int32),
            pltpu.VMEM((SUB6, H), jnp.float32),
            pltpu.VMEM((16, H), jnp.float32),
            pltpu.SemaphoreType.DMA,
        ],
    )
    return f(hb, perm_pad, srcs_pad, tb, hatom)


EC5 = 40
EPT5 = E // NW


BLK5 = 1000


def _idx_sub(dstref, srcref, o):
    dstref[pl.ds(0, 16)] = srcref[pl.ds(o, 16)]
    dstref[pl.ds(16, 16)] = srcref[pl.ds(o + 16, 16)]
    dstref[pl.ds(24, 16)] = srcref[pl.ds(o + 24, 16)]


def _edge_kernel(xnew, hb, src, dst, out, iablk, ibblk, ias, ibs, xa, xb,
                 sem, sem2):
    w = _wid()
    base_t = w * EPT5

    def block(b, _):
        bbase = base_t + b * BLK5
        pltpu.sync_copy(src.at[pl.ds(bbase, BLK5)], iablk)
        pltpu.sync_copy(dst.at[pl.ds(bbase, BLK5)], ibblk)
        def sub(c, _):
            o = c * EC5
            _idx_sub(ias, iablk, o)
            _idx_sub(ibs, ibblk, o)
            cp1 = pltpu.async_copy(xnew.at[ias], xa, sem)
            cp2 = pltpu.async_copy(hb.at[ibs], xb, sem2)
            cp1.wait()
            cp2.wait()
            for r in range(EC5):
                for f in range(8):
                    sl = pl.ds(f * 16, 16)
                    xa[r, sl] = xa[r, sl] - xb[r, sl]
            pltpu.sync_copy(xa, out.at[pl.ds(bbase + o, EC5)])
            return 0
        lax.fori_loop(0, BLK5 // EC5, sub, 0)
        return 0
    lax.fori_loop(0, EPT5 // BLK5, block, 0)


@jax.jit
def _edge_call(xnew, hb, src, dst):
    f = pl.kernel(
        _edge_kernel,
        out_type=jax.ShapeDtypeStruct((E, H), jnp.float32),
        mesh=_mesh,
        scratch_types=[
            pltpu.VMEM((BLK5,), jnp.int32),
            pltpu.VMEM((BLK5,), jnp.int32),
            pltpu.VMEM((EC5,), jnp.int32),
            pltpu.VMEM((EC5,), jnp.int32),
            pltpu.VMEM((EC5, H), jnp.float32),
            pltpu.VMEM((EC5, H), jnp.float32),
            pltpu.SemaphoreType.DMA,
            pltpu.SemaphoreType.DMA,
        ],
    )
    return f(xnew, hb, src, dst)


RC1 = 128
NCH1 = LCAP * G // RC1 // NW


def _pad_kernel(msg, pad_idx, lmax_v, mpad, idxb, rowb, lmb, sem):
    w = _wid()
    pltpu.sync_copy(lmax_v.at[pl.ds(0, 16)], lmb)
    lmax = lmb[...][0]

    def chunk(c, _):
        row0 = (c * NW + w) * RC1
        t0 = row0 // G

        @pl.when(t0 < lmax)
        def _():
            pltpu.sync_copy(pad_idx.at[pl.ds(row0, RC1)], idxb)
            pltpu.async_copy(msg.at[idxb], rowb, sem).wait()
            pltpu.sync_copy(rowb, mpad.at[pl.ds(row0, RC1)])
        return 0
    lax.fori_loop(0, NCH1, chunk, 0)


@jax.jit
def _pad_call(msg_ext, pad_idx, lmax_v):
    f = pl.kernel(
        _pad_kernel,
        out_type=jax.ShapeDtypeStruct((LCAP * G, H), jnp.float32),
        mesh=_mesh,
        scratch_types=[
            pltpu.VMEM((RC1,), jnp.int32),
            pltpu.VMEM((RC1, H), jnp.float32),
            pltpu.VMEM((16,), jnp.int32),
            pltpu.SemaphoreType.DMA,
        ],
    )
    return f(msg_ext, pad_idx, lmax_v)


def _h0_kernel(hmat, gst_pad, h0, win, rowb, h0b, sem):
    w = _wid()
    pltpu.sync_copy(gst_pad.at[pl.ds(w * 16, 24)], win)
    lo = win[pl.ds(0, 16)]
    hi = win[pl.ds(8, 16)]
    for j in range(16):
        g0 = _ext(lo, j)
        g1 = _ext(lo, j + 1) if j < 15 else _ext(hi, 8)
        g0a = (g0 // 8) * 8
        nch = (g1 - g0a + 7) // 8

        def chunk(c, acc):
            p0 = g0a + c * 8
            pltpu.sync_copy(hmat.at[pl.ds(p0, 8)], rowb)
            for i in range(8):
                p = p0 + i
                valid = jnp.logical_and(p >= g0, p < g1)
                acc = tuple(
                    jnp.maximum(acc[f],
                                jnp.where(valid, rowb[i, pl.ds(f * 16, 16)],
                                          jnp.float32(NEG)))
                    for f in range(8))
            return acc
        acc0 = tuple(jnp.full((16,), NEG, jnp.float32) for _ in range(8))
        acc = lax.fori_loop(0, nch, chunk, acc0)
        for f in range(8):
            h0b[j, pl.ds(f * 16, 16)] = jnp.where(acc[f] < -1e37,
                                                  jnp.float32(0.0), acc[f])
    pltpu.sync_copy(h0b, h0.at[pl.ds(w * 16, 16)])


@jax.jit
def _h0_call(hmat, gst_pad):
    f = pl.kernel(
        _h0_kernel,
        out_type=jax.ShapeDtypeStruct((G, H), jnp.float32),
        mesh=_mesh,
        scratch_types=[
            pltpu.VMEM((24,), jnp.int32),
            pltpu.VMEM((8, H), jnp.float32),
            pltpu.VMEM((16, H), jnp.float32),
            pltpu.SemaphoreType.DMA,
        ],
    )
    return f(hmat, gst_pad)


OROWS = LCAP * G + N + 16
RPT2 = 320


def _unpad_kernel(of, ob, oidx, outf, outb, idxb, ra, rb, sem):
    w = _wid()

    def chunk(c, _):
        row0 = w * RPT2 + c * 16

        @pl.when(row0 < N)
        def _():
            pltpu.sync_copy(oidx.at[pl.ds(row0, 16)], idxb)
            pltpu.async_copy(of.at[idxb], ra, sem).wait()
            pltpu.async_copy(ob.at[idxb], rb, sem).wait()
            pltpu.sync_copy(ra, outf.at[pl.ds(row0, 16)])
            pltpu.sync_copy(rb, outb.at[pl.ds(row0, 16)])
        return 0
    lax.fori_loop(0, RPT2 // 16, chunk, 0)


@jax.jit
def _unpad_call(of, ob, oidx):
    f = pl.kernel(
        _unpad_kernel,
        out_type=(jax.ShapeDtypeStruct((N, H), jnp.float32),
                  jax.ShapeDtypeStruct((N, H), jnp.float32)),
        mesh=_mesh,
        scratch_types=[
            pltpu.VMEM((16,), jnp.int32),
            pltpu.VMEM((16, H), jnp.float32),
            pltpu.VMEM((16, H), jnp.float32),
            pltpu.SemaphoreType.DMA,
        ],
    )
    return f(of, ob, oidx)


def _mm_relu_body(a_ref, w_ref, b_ref, o_ref):
    o_ref[...] = jax.nn.relu(
        jnp.dot(a_ref[...], w_ref[...], preferred_element_type=jnp.float32)
        + b_ref[...])


def _mm_relu(a, w, b, blk):
    m, k = a.shape
    h = w.shape[1]
    return pl.pallas_call(
        _mm_relu_body,
        grid=(m // blk,),
        in_specs=[
            pl.BlockSpec((blk, k), lambda i: (i, 0)),
            pl.BlockSpec((k, h), lambda i: (0, 0)),
            pl.BlockSpec((1, h), lambda i: (0, 0)),
        ],
        out_specs=pl.BlockSpec((blk, h), lambda i: (i, 0)),
        out_shape=jax.ShapeDtypeStruct((m, h), jnp.float32),
    )(a, w, b.reshape(1, h))


def _readout_body(am_ref, ha_ref, xp_ref, w1, w2, w3, bl, gb, h_ref, msg_ref):
    hv = (jnp.dot(am_ref[...], w1[...], preferred_element_type=jnp.float32)
          + jnp.dot(ha_ref[...], w2[...], preferred_element_type=jnp.float32)
          + jnp.dot(xp_ref[...], w3[...], preferred_element_type=jnp.float32)
          + bl[...])
    h_ref[...] = hv
    msg_ref[...] = jax.nn.relu(hv + gb[...])


def _readout(am, ha, xp, W_l, b_l, gru_bias):
    blk = 1000
    w1, w2, w3 = W_l[:H], W_l[H:2 * H], W_l[2 * H:]
    return pl.pallas_call(
        _readout_body,
        grid=(N // blk,),
        in_specs=[
            pl.BlockSpec((blk, H), lambda i: (i, 0)),
            pl.BlockSpec((blk, H), lambda i: (i, 0)),
            pl.BlockSpec((blk, H), lambda i: (i, 0)),
            pl.BlockSpec((H, H), lambda i: (0, 0)),
            pl.BlockSpec((H, H), lambda i: (0, 0)),
            pl.BlockSpec((H, H), lambda i: (0, 0)),
            pl.BlockSpec((1, H), lambda i: (0, 0)),
            pl.BlockSpec((1, H), lambda i: (0, 0)),
        ],
        out_specs=[pl.BlockSpec((blk, H), lambda i: (i, 0)),
                   pl.BlockSpec((blk, H), lambda i: (i, 0))],
        out_shape=[jax.ShapeDtypeStruct((N, H), jnp.float32),
                   jax.ShapeDtypeStruct((N, H), jnp.float32)],
    )(am, ha, xp, w1, w2, w3, b_l.reshape(1, H), gru_bias.reshape(1, H))


def _final_body(a_ref, b_ref, w1, w2, bo, o_ref):
    o_ref[...] = jax.nn.relu(
        jnp.dot(a_ref[...], w1[...], preferred_element_type=jnp.float32)
        + jnp.dot(b_ref[...], w2[...], preferred_element_type=jnp.float32)
        + bo[...])


def _final(outf, outb, W_o, b_o):
    blk = 1000
    return pl.pallas_call(
        _final_body,
        grid=(N // blk,),
        in_specs=[
            pl.BlockSpec((blk, H), lambda i: (i, 0)),
            pl.BlockSpec((blk, H), lambda i: (i, 0)),
            pl.BlockSpec((H, H), lambda i: (0, 0)),
            pl.BlockSpec((H, H), lambda i: (0, 0)),
            pl.BlockSpec((1, H), lambda i: (0, 0)),
        ],
        out_specs=pl.BlockSpec((blk, H), lambda i: (i, 0)),
        out_shape=jax.ShapeDtypeStruct((N, H), jnp.float32),
    )(outf, outb, W_o[:H], W_o[H:], b_o.reshape(1, H))


def _gru_body(lmax_ref, starts_ref, counts_ref, mpad, msg_ext, wih, whh,
              bih, bhh, h0_ref, outf, outb, h_ref, xt_ref, sem0, sem1):
    lmax = lmax_ref[0]

    for d in (0, 1):
        out_ref = outf if d == 0 else outb
        h_ref[...] = h0_ref[...]

        def step(i, _):
            t = i if d == 0 else lmax - 1 - i

            @pl.when(t < LCAP)
            def _():
                cp = pltpu.make_async_copy(
                    mpad.at[pl.ds(t * G, G)], xt_ref, sem0)
                cp.start()
                cp.wait()

            @pl.when(t >= LCAP)
            def _():
                def gb(g, _):
                    idx = jnp.where(t < counts_ref[g],
                                    starts_ref[g] + t, N)
                    cp = pltpu.make_async_copy(
                        msg_ext.at[pl.ds(idx, 1)], xt_ref.at[pl.ds(g, 1)],
                        sem0)
                    cp.start()
                    cp.wait()
                    return 0
                lax.fori_loop(0, G, gb, 0)

            xt = xt_ref[...]
            hprev = h_ref[...]
            gi = jnp.dot(xt, wih[d], preferred_element_type=jnp.float32) \
                + bih[d]
            gh = jnp.dot(hprev, whh[d], preferred_element_type=jnp.float32) \
                + bhh[d]
            r = jax.nn.sigmoid(gi[:, :H] + gh[:, :H])
            z = jax.nn.sigmoid(gi[:, H:2 * H] + gh[:, H:2 * H])
            n = jnp.tanh(gi[:, 2 * H:] + r * gh[:, 2 * H:])
            hn = (1.0 - z) * n + z * hprev
            h_ref[...] = hn

            @pl.when(t < LCAP)
            def _():
                cp = pltpu.make_async_copy(
                    h_ref, out_ref.at[pl.ds(t * G, G)], sem1)
                cp.start()
                cp.wait()

            @pl.when(t >= LCAP)
            def _():
                def sb(g, _):
                    node = jnp.where(t < counts_ref[g],
                                     starts_ref[g] + t, N)
                    cp = pltpu.make_async_copy(
                        h_ref.at[pl.ds(g, 1)],
                        out_ref.at[pl.ds(LCAP * G + node, 1)], sem1)
                    cp.start()
                    cp.wait()
                    return 0
                lax.fori_loop(0, G, sb, 0)
            return 0
        lax.fori_loop(0, lmax, step, 0)


def _gru(lmax_v, gstarts, counts, mpad, msg_ext, wihT, whhT, bih, bhh, h0):
    return pl.pallas_call(
        _gru_body,
        in_specs=[
            pl.BlockSpec(memory_space=pltpu.SMEM),
            pl.BlockSpec(memory_space=pltpu.SMEM),
            pl.BlockSpec(memory_space=pltpu.SMEM),
            pl.BlockSpec(memory_space=pl.ANY),
            pl.BlockSpec(memory_space=pl.ANY),
            pl.BlockSpec(memory_space=pltpu.VMEM),
            pl.BlockSpec(memory_space=pltpu.VMEM),
            pl.BlockSpec(memory_space=pltpu.VMEM),
            pl.BlockSpec(memory_space=pltpu.VMEM),
            pl.BlockSpec(memory_space=pltpu.VMEM),
        ],
        out_specs=[pl.BlockSpec(memory_space=pl.ANY),
                   pl.BlockSpec(memory_space=pl.ANY)],
        out_shape=[jax.ShapeDtypeStruct((OROWS, H), jnp.float32),
                   jax.ShapeDtypeStruct((OROWS, H), jnp.float32)],
        scratch_shapes=[
            pltpu.VMEM((G, H), jnp.float32),
            pltpu.VMEM((G, H), jnp.float32),
            pltpu.SemaphoreType.DMA,
            pltpu.SemaphoreType.DMA,
        ],
    )(lmax_v, gstarts, counts, mpad, msg_ext, wihT, whhT, bih, bhh, h0)


def kernel(x, edge_index, edge_attr, batch, W_a, b_a, W_b, b_b, Ws, bs, W_l,
           b_l, gru_bias, Wih, Whh, bih, bhh, W_o, b_o):
    src = edge_index[0]
    dst = edge_index[1]

    perm = jnp.argsort(src).astype(jnp.int32)
    src_sorted = src[perm]
    bounds = jnp.searchsorted(
        src_sorted, jnp.arange(NW + 1, dtype=jnp.int32) * NT6
    ).astype(jnp.int32)
    tb = jnp.zeros((NW, 16), jnp.int32)
    tb = tb.at[:, 0].set(bounds[:-1]).at[:, 1].set(bounds[1:]).reshape(-1)
    perm_pad = jnp.concatenate([perm, jnp.zeros((BLK6,), jnp.int32)])
    srcs_pad = jnp.concatenate(
        [src_sorted, jnp.full((BLK6,), 1 << 30, jnp.int32)])

    gstarts = jnp.searchsorted(batch, jnp.arange(G + 1, dtype=jnp.int32)
                               ).astype(jnp.int32)
    counts = gstarts[1:] - gstarts[:-1]
    lmax = jnp.max(counts)
    lmax_v = jnp.full((16,), lmax, jnp.int32)
    gst_pad = jnp.concatenate([gstarts, jnp.full((15,), N, jnp.int32)])

    tt = jnp.arange(LCAP, dtype=jnp.int32)[:, None]
    gg = jnp.arange(G, dtype=jnp.int32)[None, :]
    pad_idx = jnp.where(tt < counts[None, :], gstarts[None, :-1] + tt,
                        N).reshape(-1).astype(jnp.int32)

    node = jnp.arange(N, dtype=jnp.int32)
    is_start = jnp.concatenate(
        [jnp.ones((1,), jnp.bool_), batch[1:] != batch[:-1]])
    my_start = lax.cummax(jnp.where(is_start, node, 0))
    tnode = node - my_start
    oidx = jnp.where(tnode < LCAP, tnode * G + batch,
                     LCAP * G + node).astype(jnp.int32)

    x_proj = _mm_relu(x, W_a, b_a, 1000)
    h_bond = _mm_relu(edge_attr, W_b, b_b, 2000)

    h_atom = x_proj
    for l in range(NL - 1):
        x_new = _seg_call(h_bond, perm_pad, srcs_pad, tb, h_atom)
        bond_embed = _edge_call(x_new, h_bond, src, dst)
        h_bond = _mm_relu(bond_embed, Ws[l], bs[l], 2000)
        h_atom = x_new
    aggr = _seg_call(h_bond, perm_pad, srcs_pad, tb, h_atom)

    h, message = _readout(aggr, h_atom, x_proj, W_l, b_l, gru_bias)
    msg_ext = jnp.concatenate([message, jnp.zeros((16, H), jnp.float32)])

    h0 = _h0_call(h, gst_pad)
    mpad = _pad_call(msg_ext, pad_idx, lmax_v)
    of, ob = _gru(lmax.reshape(1), gstarts[:-1], counts, mpad, msg_ext,
                  jnp.transpose(Wih, (0, 2, 1)), jnp.transpose(Whh, (0, 2, 1)),
                  bih[:, None, :], bhh[:, None, :], h0)
    out_f, out_b = _unpad_call(of, ob, oidx)
    return _final(out_f, out_b, W_o, b_o)
